# baseline (device time: 26435 ns/iter reference)
import jax
import jax.numpy as jnp
from jax import lax
from jax.experimental import pallas as pl
from jax.experimental.pallas import tpu as pltpu

_BLOCK_M = 512
_EPS = 1e-5


def _partial_body(x_ref, dy_ref, out_ref):
    xb = x_ref[:, :]
    dyb = dy_ref[:, :]
    mu = jnp.mean(xb, axis=1, keepdims=True)
    xc = xb - mu
    var = jnp.mean(xc * xc, axis=1, keepdims=True)
    xhat = xc * lax.rsqrt(var + _EPS)
    part = jnp.stack([jnp.sum(dyb * xhat, axis=0), jnp.sum(dyb, axis=0)])

    @pl.when(pl.program_id(0) == 0)
    def _():
        out_ref[:, :] = part

    @pl.when(pl.program_id(0) != 0)
    def _():
        out_ref[:, :] = out_ref[:, :] + part


def _exchange_body(p_ref, out_ref, recv_ref, send_sem, recv_sem):
    my_x = lax.axis_index("x")
    my_y = lax.axis_index("y")
    my_z = lax.axis_index("z")
    partner = (1 - my_x, my_y, my_z)

    barrier = pltpu.get_barrier_semaphore()
    pl.semaphore_signal(
        barrier, inc=1, device_id=partner, device_id_type=pl.DeviceIdType.MESH
    )
    pl.semaphore_wait(barrier, 1)

    rdma = pltpu.make_async_remote_copy(
        src_ref=p_ref,
        dst_ref=recv_ref,
        send_sem=send_sem,
        recv_sem=recv_sem,
        device_id=partner,
        device_id_type=pl.DeviceIdType.MESH,
    )
    rdma.start()
    rdma.wait()
    out_ref[:, :] = p_ref[:, :] + recv_ref[:, :]


def kernel(x, dy, gamma):
    del gamma
    m, d = x.shape
    partial = pl.pallas_call(
        _partial_body,
        grid=(m // _BLOCK_M,),
        in_specs=[
            pl.BlockSpec((_BLOCK_M, d), lambda i: (i, 0)),
            pl.BlockSpec((_BLOCK_M, d), lambda i: (i, 0)),
        ],
        out_specs=pl.BlockSpec((2, d), lambda i: (0, 0)),
        out_shape=jax.ShapeDtypeStruct((2, d), jnp.float32),
    )(x, dy)

    return pl.pallas_call(
        _exchange_body,
        out_shape=jax.ShapeDtypeStruct((2, d), jnp.float32),
        in_specs=[pl.BlockSpec(memory_space=pltpu.VMEM)],
        out_specs=pl.BlockSpec(memory_space=pltpu.VMEM),
        scratch_shapes=[
            pltpu.VMEM((2, d), jnp.float32),
            pltpu.SemaphoreType.DMA,
            pltpu.SemaphoreType.DMA,
        ],
        compiler_params=pltpu.CompilerParams(collective_id=0),
    )(partial)
